# baseline (speedup 1.0000x reference)
import jax
import jax.numpy as jnp
from jax import lax
from jax.experimental import pallas as pl
from jax.experimental.pallas import tpu as pltpu
from jax.experimental.pallas import tpu_sc as plsc

N_P = 5000
NODE_SIZE = 10000
FT = 256
HID = 256
OUT = 256
E = 80000

NC, NS, L = 2, 16, 16
SEG = FT // L
RT = 320
ACC_ROWS = NS * RT
EC = 1024
EPAD = 81920
CAP = 64
FLUSH_AT = CAP - L


def _spmm_body(rows_hbm, cols_hbm, w_hbm, table_hbm, zeros_hbm, out_hbm,
               rbuf, cbuf, wbuf, cstage0, wstage0, rstage0, cstage1, wstage1,
               rstage1, xbuf0, xbuf1, acc, sem0, sem1):
  c = lax.axis_index("c")
  s = lax.axis_index("s")
  lo = s * RT
  cstages = (cstage0, cstage1)
  wstages = (wstage0, wstage1)
  rstages = (rstage0, rstage1)
  xbufs = (xbuf0, xbuf1)
  sems = (sem0, sem1)

  pltpu.sync_copy(zeros_hbm, acc)
  zero_i = jnp.zeros((L,), jnp.int32)
  zero_f = jnp.zeros((L,), jnp.float32)
  for b in range(2):
    for g in range(CAP // L):
      cstages[b][pl.ds(g * L, L)] = zero_i
      rstages[b][pl.ds(g * L, L)] = zero_i
      wstages[b][pl.ds(g * L, L)] = zero_f

  def start(b):
    pltpu.async_copy(table_hbm.at[cstages[b]], xbufs[b], sems[b])

  def finish(b):
    pltpu.make_async_copy(table_hbm.at[cstages[b]], xbufs[b], sems[b]).wait()

    def fgroup(g, carry):
      w16 = wstages[b][pl.ds(g * L, L)]
      r16 = rstages[b][pl.ds(g * L, L)]
      for l in range(L):
        w = w16[l]
        r = r16[l]
        j = g * L + l

        @plsc.parallel_loop(0, SEG, unroll=8)
        def _(si):
          sl = pl.ds(si * L, L)
          plsc.addupdate(acc.at[r, sl], xbufs[b][j, sl] * w)

      wstages[b][pl.ds(g * L, L)] = zero_f
      return carry

    lax.fori_loop(0, CAP // L, fgroup, 0)

  def chunk(ch, st):
    base = ch * EC
    pltpu.sync_copy(rows_hbm.at[c, pl.ds(base, EC)], rbuf)
    pltpu.sync_copy(cols_hbm.at[c, pl.ds(base, EC)], cbuf)
    pltpu.sync_copy(w_hbm.at[c, pl.ds(base, EC)], wbuf)

    def group(g, st2):
      cnt, bank, pending = st2
      row16 = rbuf[pl.ds(g * L, L)]
      m = (row16 >= lo) & (row16 < lo + RT)
      inc = plsc.cumsum(jnp.where(m, 1, 0))
      pos = cnt + inc - 1
      col16 = cbuf[pl.ds(g * L, L)]
      w16 = wbuf[pl.ds(g * L, L)]
      r16 = row16 - lo
      for b in range(2):
        @pl.when(bank == b)
        def _():
          plsc.store_scatter(cstages[b], [pos], col16, mask=m)
          plsc.store_scatter(wstages[b], [pos], w16, mask=m)
          plsc.store_scatter(rstages[b], [pos], r16, mask=m)
      cnt = cnt + inc[L - 1]
      trig = cnt >= FLUSH_AT
      for b in range(2):
        @pl.when(trig & (bank == b))
        def _():
          start(b)

          @pl.when(pending == 1)
          def _():
            finish(1 - b)

      cnt = jnp.where(trig, 0, cnt)
      pending = jnp.where(trig, 1, pending)
      bank = jnp.where(trig, 1 - bank, bank)
      return (cnt, bank, pending)

    return lax.fori_loop(0, EC // L, group, st)

  cnt, bank, pending = lax.fori_loop(
      0, EPAD // EC, chunk,
      (jnp.int32(0), jnp.int32(0), jnp.int32(0)))

  for b in range(2):
    @pl.when((pending == 1) & (bank == 1 - b))
    def _():
      finish(b)
  for b in range(2):
    @pl.when((cnt > 0) & (bank == b))
    def _():
      start(b)
      finish(b)

  pltpu.sync_copy(acc, out_hbm.at[c, pl.ds(lo, RT)])


@jax.jit
def _spmm2(table, rows2, cols2, w2, zeros):
  mesh = plsc.VectorSubcoreMesh(core_axis_name="c", subcore_axis_name="s")
  return pl.kernel(
      _spmm_body,
      out_type=jax.ShapeDtypeStruct((NC, ACC_ROWS, FT), jnp.float32),
      mesh=mesh,
      compiler_params=pltpu.CompilerParams(needs_layout_passes=False),
      scratch_types=[
          pltpu.VMEM((EC,), jnp.int32),
          pltpu.VMEM((EC,), jnp.int32),
          pltpu.VMEM((EC,), jnp.float32),
          pltpu.VMEM((CAP,), jnp.int32),
          pltpu.VMEM((CAP,), jnp.float32),
          pltpu.VMEM((CAP,), jnp.int32),
          pltpu.VMEM((CAP,), jnp.int32),
          pltpu.VMEM((CAP,), jnp.float32),
          pltpu.VMEM((CAP,), jnp.int32),
          pltpu.VMEM((CAP, FT), jnp.float32),
          pltpu.VMEM((CAP, FT), jnp.float32),
          pltpu.VMEM((RT, FT), jnp.float32),
          pltpu.SemaphoreType.DMA,
          pltpu.SemaphoreType.DMA,
      ],
  )(rows2, cols2, w2, table, zeros)


def _gcn_matmul_body(mn_ref, w_ref, out_ref):
  out_ref[0] = jnp.maximum(
      jnp.dot(mn_ref[0], w_ref[0], preferred_element_type=jnp.float32), 0.0)


BR = 1280


@jax.jit
def _gcn_layer(mn, w_stack):
  return pl.pallas_call(
      _gcn_matmul_body,
      grid=(NC, ACC_ROWS // BR),
      in_specs=[
          pl.BlockSpec((1, BR, FT), lambda c, i: (c, i, 0)),
          pl.BlockSpec((1, FT, HID), lambda c, i: (c, 0, 0)),
      ],
      out_specs=pl.BlockSpec((1, BR, HID), lambda c, i: (c, i, 0)),
      out_shape=jax.ShapeDtypeStruct((NC, ACC_ROWS, HID), jnp.float32),
  )(mn, w_stack)


def _final_body(mn2_ref, feat_ref, w1_ref, wfa_ref, wfb_ref, b_ref, out_ref):
  v = jnp.maximum(
      jnp.dot(mn2_ref[0], w1_ref[0], preferred_element_type=jnp.float32), 0.0)
  out_ref[...] = (
      jnp.dot(v, wfa_ref[0], preferred_element_type=jnp.float32)
      + jnp.dot(feat_ref[...], wfb_ref[0], preferred_element_type=jnp.float32)
      + b_ref[0])


FR = 1000


@jax.jit
def _final_layer(mn2, features, w1_stack, wfa_stack, wfb_stack, b_stack):
  nb = N_P // FR
  return pl.pallas_call(
      _final_body,
      grid=(NC * nb,),
      in_specs=[
          pl.BlockSpec((1, FR, HID), lambda i: (i // nb, i % nb, 0)),
          pl.BlockSpec((FR, FT), lambda i: (i, 0)),
          pl.BlockSpec((1, HID, HID), lambda i: (i // nb, 0, 0)),
          pl.BlockSpec((1, HID, OUT), lambda i: (i // nb, 0, 0)),
          pl.BlockSpec((1, FT, OUT), lambda i: (i // nb, 0, 0)),
          pl.BlockSpec((1, 1, OUT), lambda i: (i // nb, 0, 0)),
      ],
      out_specs=pl.BlockSpec((FR, OUT), lambda i: (i, 0)),
      out_shape=jax.ShapeDtypeStruct((NODE_SIZE, OUT), jnp.float32),
  )(mn2, features, w1_stack, wfa_stack, wfb_stack, b_stack)


def kernel(features, edge_index_p, edge_weight_p, edge_index_a, edge_weight_a,
           idx_p, idx_a, W0_pa, W0_ap, W1_pa, W1_ap, Wfc_p, bfc_p, Wfc_a,
           bfc_a):
  pad = EPAD - E
  rows2 = jnp.stack([
      jnp.pad(edge_index_p[0], (0, pad)),
      jnp.pad(edge_index_a[0], (0, pad)),
  ])
  cols_l0 = jnp.stack([
      jnp.pad(edge_index_p[1] + N_P, (0, pad)),
      jnp.pad(edge_index_a[1], (0, pad)),
  ])
  cols_l1 = jnp.stack([
      jnp.pad(edge_index_p[1] + ACC_ROWS, (0, pad)),
      jnp.pad(edge_index_a[1], (0, pad)),
  ])
  w2 = jnp.stack([
      jnp.pad(edge_weight_p, (0, pad)),
      jnp.pad(edge_weight_a, (0, pad)),
  ])
  zeros = jnp.zeros((RT, FT), jnp.float32)

  mn = _spmm2(features, rows2, cols_l0, w2, zeros)
  w0_stack = jnp.stack([W0_pa, W0_ap])
  embs1 = _gcn_layer(mn, w0_stack)
  mn2 = _spmm2(embs1.reshape(NC * ACC_ROWS, HID), rows2, cols_l1, w2, zeros)
  w1_stack = jnp.stack([W1_pa, W1_ap])
  wfa_stack = jnp.stack([Wfc_p[:HID], Wfc_a[:HID]])
  wfb_stack = jnp.stack([Wfc_p[HID:], Wfc_a[HID:]])
  b_stack = jnp.stack([bfc_p, bfc_a]).reshape(NC, 1, OUT)
  return _final_layer(mn2, features, w1_stack, wfa_stack, wfb_stack, b_stack)


# speedup vs baseline: 1.7599x; 1.7599x over previous
import jax
import jax.numpy as jnp
from jax import lax
from jax.experimental import pallas as pl
from jax.experimental.pallas import tpu as pltpu
from jax.experimental.pallas import tpu_sc as plsc

N_P = 5000
NODE_SIZE = 10000
FT = 256
HID = 256
OUT = 256
E = 80000

NC, NS, L = 2, 16, 16
SEG = FT // L
RT = 320
ACC_ROWS = NS * RT
EC = 1024
EPAD = 81920
CAP = 128
FLUSH_AT = CAP - L


def _spmm_body(rows_hbm, cols_hbm, w_hbm, table_hbm, zeros_hbm, out_hbm,
               rbuf, cbuf, wbuf, cstage, wstage, rstage, xbuf, acc, sem):
  c = lax.axis_index("c")
  s = lax.axis_index("s")
  lo = s * RT

  pltpu.sync_copy(zeros_hbm, acc)
  zero_i = jnp.zeros((L,), jnp.int32)
  zero_f = jnp.zeros((L,), jnp.float32)
  for g in range(CAP // L):
    cstage[pl.ds(g * L, L)] = zero_i
    rstage[pl.ds(g * L, L)] = zero_i
    wstage[pl.ds(g * L, L)] = zero_f

  def flush():
    pltpu.async_copy(table_hbm.at[cstage], xbuf, sem).wait()

    def fgroup(g, carry):
      w16 = wstage[pl.ds(g * L, L)]
      r16 = rstage[pl.ds(g * L, L)]
      for l in range(L):
        w = w16[l]
        r = r16[l]
        j = g * L + l

        @plsc.parallel_loop(0, SEG, unroll=8)
        def _(si):
          sl = pl.ds(si * L, L)
          plsc.addupdate(acc.at[r, sl], xbuf[j, sl] * w)
      wstage[pl.ds(g * L, L)] = zero_f
      return carry

    lax.fori_loop(0, CAP // L, fgroup, 0)

  def chunk(ch, cnt):
    base = ch * EC
    pltpu.sync_copy(rows_hbm.at[c, pl.ds(base, EC)], rbuf)
    pltpu.sync_copy(cols_hbm.at[c, pl.ds(base, EC)], cbuf)
    pltpu.sync_copy(w_hbm.at[c, pl.ds(base, EC)], wbuf)

    def group(g, cnt2):
      row16 = rbuf[pl.ds(g * L, L)]
      m = (row16 >= lo) & (row16 < lo + RT)
      inc = plsc.cumsum(jnp.where(m, 1, 0))
      pos = cnt2 + inc - 1
      plsc.store_scatter(cstage, [pos], cbuf[pl.ds(g * L, L)], mask=m)
      plsc.store_scatter(wstage, [pos], wbuf[pl.ds(g * L, L)], mask=m)
      plsc.store_scatter(rstage, [pos], row16 - lo, mask=m)
      cnt2 = cnt2 + inc[L - 1]

      @pl.when(cnt2 >= FLUSH_AT)
      def _():
        flush()

      return jnp.where(cnt2 >= FLUSH_AT, 0, cnt2)

    return lax.fori_loop(0, EC // L, group, cnt)

  cnt = lax.fori_loop(0, EPAD // EC, chunk, jnp.int32(0))

  @pl.when(cnt > 0)
  def _():
    flush()

  pltpu.sync_copy(acc, out_hbm.at[c, pl.ds(lo, RT)])


@jax.jit
def _spmm2(table, rows2, cols2, w2, zeros):
  mesh = plsc.VectorSubcoreMesh(core_axis_name="c", subcore_axis_name="s")
  return pl.kernel(
      _spmm_body,
      out_type=jax.ShapeDtypeStruct((NC, ACC_ROWS, FT), jnp.float32),
      mesh=mesh,
      compiler_params=pltpu.CompilerParams(needs_layout_passes=False),
      scratch_types=[
          pltpu.VMEM((EC,), jnp.int32),
          pltpu.VMEM((EC,), jnp.int32),
          pltpu.VMEM((EC,), jnp.float32),
          pltpu.VMEM((CAP,), jnp.int32),
          pltpu.VMEM((CAP,), jnp.float32),
          pltpu.VMEM((CAP,), jnp.int32),
          pltpu.VMEM((CAP, FT), jnp.float32),
          pltpu.VMEM((RT, FT), jnp.float32),
          pltpu.SemaphoreType.DMA,
      ],
  )(rows2, cols2, w2, table, zeros)


def _gcn_matmul_body(mn_ref, w_ref, out_ref):
  out_ref[0] = jnp.maximum(
      jnp.dot(mn_ref[0], w_ref[0], preferred_element_type=jnp.float32), 0.0)


BR = 1280


@jax.jit
def _gcn_layer(mn, w_stack):
  return pl.pallas_call(
      _gcn_matmul_body,
      grid=(NC, ACC_ROWS // BR),
      in_specs=[
          pl.BlockSpec((1, BR, FT), lambda c, i: (c, i, 0)),
          pl.BlockSpec((1, FT, HID), lambda c, i: (c, 0, 0)),
      ],
      out_specs=pl.BlockSpec((1, BR, HID), lambda c, i: (c, i, 0)),
      out_shape=jax.ShapeDtypeStruct((NC, ACC_ROWS, HID), jnp.float32),
  )(mn, w_stack)


def _final_body(mn2_ref, feat_ref, w1_ref, wfa_ref, wfb_ref, b_ref, out_ref):
  v = jnp.maximum(
      jnp.dot(mn2_ref[0], w1_ref[0], preferred_element_type=jnp.float32), 0.0)
  out_ref[...] = (
      jnp.dot(v, wfa_ref[0], preferred_element_type=jnp.float32)
      + jnp.dot(feat_ref[...], wfb_ref[0], preferred_element_type=jnp.float32)
      + b_ref[0])


FR = 1000


@jax.jit
def _final_layer(mn2, features, w1_stack, wfa_stack, wfb_stack, b_stack):
  nb = N_P // FR
  return pl.pallas_call(
      _final_body,
      grid=(NC * nb,),
      in_specs=[
          pl.BlockSpec((1, FR, HID), lambda i: (i // nb, i % nb, 0)),
          pl.BlockSpec((FR, FT), lambda i: (i, 0)),
          pl.BlockSpec((1, HID, HID), lambda i: (i // nb, 0, 0)),
          pl.BlockSpec((1, HID, OUT), lambda i: (i // nb, 0, 0)),
          pl.BlockSpec((1, FT, OUT), lambda i: (i // nb, 0, 0)),
          pl.BlockSpec((1, 1, OUT), lambda i: (i // nb, 0, 0)),
      ],
      out_specs=pl.BlockSpec((FR, OUT), lambda i: (i, 0)),
      out_shape=jax.ShapeDtypeStruct((NODE_SIZE, OUT), jnp.float32),
  )(mn2, features, w1_stack, wfa_stack, wfb_stack, b_stack)


def kernel(features, edge_index_p, edge_weight_p, edge_index_a, edge_weight_a,
           idx_p, idx_a, W0_pa, W0_ap, W1_pa, W1_ap, Wfc_p, bfc_p, Wfc_a,
           bfc_a):
  pad = EPAD - E
  rows2 = jnp.stack([
      jnp.pad(edge_index_p[0], (0, pad)),
      jnp.pad(edge_index_a[0], (0, pad)),
  ])
  cols_l0 = jnp.stack([
      jnp.pad(edge_index_p[1] + N_P, (0, pad)),
      jnp.pad(edge_index_a[1], (0, pad)),
  ])
  cols_l1 = jnp.stack([
      jnp.pad(edge_index_p[1] + ACC_ROWS, (0, pad)),
      jnp.pad(edge_index_a[1], (0, pad)),
  ])
  w2 = jnp.stack([
      jnp.pad(edge_weight_p, (0, pad)),
      jnp.pad(edge_weight_a, (0, pad)),
  ])
  zeros = jnp.zeros((RT, FT), jnp.float32)

  mn = _spmm2(features, rows2, cols_l0, w2, zeros)
  w0_stack = jnp.stack([W0_pa, W0_ap])
  embs1 = _gcn_layer(mn, w0_stack)
  mn2 = _spmm2(embs1.reshape(NC * ACC_ROWS, HID), rows2, cols_l1, w2, zeros)
  w1_stack = jnp.stack([W1_pa, W1_ap])
  wfa_stack = jnp.stack([Wfc_p[:HID], Wfc_a[:HID]])
  wfb_stack = jnp.stack([Wfc_p[HID:], Wfc_a[HID:]])
  b_stack = jnp.stack([bfc_p, bfc_a]).reshape(NC, 1, OUT)
  return _final_layer(mn2, features, w1_stack, wfa_stack, wfb_stack, b_stack)


# speedup vs baseline: 1.7771x; 1.0098x over previous
import jax
import jax.numpy as jnp
from jax import lax
from jax.experimental import pallas as pl
from jax.experimental.pallas import tpu as pltpu
from jax.experimental.pallas import tpu_sc as plsc

N_P = 5000
NODE_SIZE = 10000
FT = 256
HID = 256
OUT = 256
E = 80000

NC, NS, L = 2, 16, 16
SEG = FT // L
RT = 320
ACC_ROWS = NS * RT
EC = 1024
EPAD = 81920
CAP = 128
FLUSH_AT = CAP - L


def _spmm_body(rows_hbm, cols_hbm, w_hbm, table_hbm, zeros_hbm, out_hbm,
               rbuf, cbuf, wbuf, cstage, wstage, rstage, xbuf, acc, sem):
  c = lax.axis_index("c")
  s = lax.axis_index("s")
  lo = s * RT

  pltpu.sync_copy(zeros_hbm, acc)
  zero_i = jnp.zeros((L,), jnp.int32)
  zero_f = jnp.zeros((L,), jnp.float32)
  for g in range(CAP // L):
    cstage[pl.ds(g * L, L)] = zero_i
    rstage[pl.ds(g * L, L)] = zero_i
    wstage[pl.ds(g * L, L)] = zero_f

  def flush():
    pltpu.async_copy(table_hbm.at[cstage], xbuf, sem).wait()

    def fgroup(g, carry):
      w16 = wstage[pl.ds(g * L, L)]
      r16 = rstage[pl.ds(g * L, L)]
      for l in range(L):
        w = w16[l]
        r = r16[l]
        j = g * L + l

        @plsc.parallel_loop(0, SEG // 2, unroll=4)
        def _(si):
          xw = xbuf[j, pl.ds(si * L, L)]
          xb = plsc.bitcast(xw, jnp.bfloat16)
          va, vb = plsc.unpack(xb, format=plsc.PackFormat.INTERLEAVED,
                               preferred_element_type=jnp.float32)
          plsc.addupdate(acc.at[r, pl.ds(2 * si * L, L)], va * w)
          plsc.addupdate(acc.at[r, pl.ds((2 * si + 1) * L, L)], vb * w)
      wstage[pl.ds(g * L, L)] = zero_f
      return carry

    lax.fori_loop(0, CAP // L, fgroup, 0)

  def chunk(ch, cnt):
    base = ch * EC
    pltpu.sync_copy(rows_hbm.at[c, pl.ds(base, EC)], rbuf)
    pltpu.sync_copy(cols_hbm.at[c, pl.ds(base, EC)], cbuf)
    pltpu.sync_copy(w_hbm.at[c, pl.ds(base, EC)], wbuf)

    def group(g, cnt2):
      row16 = rbuf[pl.ds(g * L, L)]
      m = (row16 >= lo) & (row16 < lo + RT)
      inc = plsc.cumsum(jnp.where(m, 1, 0))
      pos = cnt2 + inc - 1
      plsc.store_scatter(cstage, [pos], cbuf[pl.ds(g * L, L)], mask=m)
      plsc.store_scatter(wstage, [pos], wbuf[pl.ds(g * L, L)], mask=m)
      plsc.store_scatter(rstage, [pos], row16 - lo, mask=m)
      cnt2 = cnt2 + inc[L - 1]

      @pl.when(cnt2 >= FLUSH_AT)
      def _():
        flush()

      return jnp.where(cnt2 >= FLUSH_AT, 0, cnt2)

    return lax.fori_loop(0, EC // L, group, cnt)

  cnt = lax.fori_loop(0, EPAD // EC, chunk, jnp.int32(0))

  @pl.when(cnt > 0)
  def _():
    flush()

  pltpu.sync_copy(acc, out_hbm.at[c, pl.ds(lo, RT)])


@jax.jit
def _spmm2(table, rows2, cols2, w2, zeros):
  mesh = plsc.VectorSubcoreMesh(core_axis_name="c", subcore_axis_name="s")
  return pl.kernel(
      _spmm_body,
      out_type=jax.ShapeDtypeStruct((NC, ACC_ROWS, FT), jnp.float32),
      mesh=mesh,
      compiler_params=pltpu.CompilerParams(needs_layout_passes=False),
      scratch_types=[
          pltpu.VMEM((EC,), jnp.int32),
          pltpu.VMEM((EC,), jnp.int32),
          pltpu.VMEM((EC,), jnp.float32),
          pltpu.VMEM((CAP,), jnp.int32),
          pltpu.VMEM((CAP,), jnp.float32),
          pltpu.VMEM((CAP,), jnp.int32),
          pltpu.VMEM((CAP, FT // 2), jnp.int32),
          pltpu.VMEM((RT, FT), jnp.float32),
          pltpu.SemaphoreType.DMA,
      ],
  )(rows2, cols2, w2, table, zeros)


def _gcn_matmul_body(mn_ref, w_ref, out_ref):
  out_ref[0] = jnp.maximum(
      jnp.dot(mn_ref[0], w_ref[0], preferred_element_type=jnp.float32), 0.0)


BR = 1280


@jax.jit
def _gcn_layer(mn, w_stack):
  return pl.pallas_call(
      _gcn_matmul_body,
      grid=(NC, ACC_ROWS // BR),
      in_specs=[
          pl.BlockSpec((1, BR, FT), lambda c, i: (c, i, 0)),
          pl.BlockSpec((1, FT, HID), lambda c, i: (c, 0, 0)),
      ],
      out_specs=pl.BlockSpec((1, BR, HID), lambda c, i: (c, i, 0)),
      out_shape=jax.ShapeDtypeStruct((NC, ACC_ROWS, HID), jnp.float32),
  )(mn, w_stack)


def _final_body(mn2_ref, feat_ref, w1_ref, wfa_ref, wfb_ref, b_ref, out_ref):
  v = jnp.maximum(
      jnp.dot(mn2_ref[0], w1_ref[0], preferred_element_type=jnp.float32), 0.0)
  out_ref[...] = (
      jnp.dot(v, wfa_ref[0], preferred_element_type=jnp.float32)
      + jnp.dot(feat_ref[...], wfb_ref[0], preferred_element_type=jnp.float32)
      + b_ref[0])


FR = 1000


@jax.jit
def _final_layer(mn2, features, w1_stack, wfa_stack, wfb_stack, b_stack):
  nb = N_P // FR
  return pl.pallas_call(
      _final_body,
      grid=(NC * nb,),
      in_specs=[
          pl.BlockSpec((1, FR, HID), lambda i: (i // nb, i % nb, 0)),
          pl.BlockSpec((FR, FT), lambda i: (i, 0)),
          pl.BlockSpec((1, HID, HID), lambda i: (i // nb, 0, 0)),
          pl.BlockSpec((1, HID, OUT), lambda i: (i // nb, 0, 0)),
          pl.BlockSpec((1, FT, OUT), lambda i: (i // nb, 0, 0)),
          pl.BlockSpec((1, 1, OUT), lambda i: (i // nb, 0, 0)),
      ],
      out_specs=pl.BlockSpec((FR, OUT), lambda i: (i, 0)),
      out_shape=jax.ShapeDtypeStruct((NODE_SIZE, OUT), jnp.float32),
  )(mn2, features, w1_stack, wfa_stack, wfb_stack, b_stack)


def kernel(features, edge_index_p, edge_weight_p, edge_index_a, edge_weight_a,
           idx_p, idx_a, W0_pa, W0_ap, W1_pa, W1_ap, Wfc_p, bfc_p, Wfc_a,
           bfc_a):
  pad = EPAD - E
  rows2 = jnp.stack([
      jnp.pad(edge_index_p[0], (0, pad)),
      jnp.pad(edge_index_a[0], (0, pad)),
  ])
  cols_l0 = jnp.stack([
      jnp.pad(edge_index_p[1] + N_P, (0, pad)),
      jnp.pad(edge_index_a[1], (0, pad)),
  ])
  cols_l1 = jnp.stack([
      jnp.pad(edge_index_p[1] + ACC_ROWS, (0, pad)),
      jnp.pad(edge_index_a[1], (0, pad)),
  ])
  w2 = jnp.stack([
      jnp.pad(edge_weight_p, (0, pad)),
      jnp.pad(edge_weight_a, (0, pad)),
  ])
  zeros = jnp.zeros((RT, FT), jnp.float32)

  def to_bf16_interleaved(t):
    n = t.shape[0]
    b = (t.reshape(n, FT // 32, 2, L).transpose(0, 1, 3, 2)
         .reshape(n, FT // 2, 2).astype(jnp.bfloat16))
    return jax.lax.bitcast_convert_type(b, jnp.int32)

  mn = _spmm2(to_bf16_interleaved(features), rows2, cols_l0, w2, zeros)
  w0_stack = jnp.stack([W0_pa, W0_ap])
  embs1 = _gcn_layer(mn, w0_stack)
  mn2 = _spmm2(to_bf16_interleaved(embs1.reshape(NC * ACC_ROWS, HID)),
               rows2, cols_l1, w2, zeros)
  w1_stack = jnp.stack([W1_pa, W1_ap])
  wfa_stack = jnp.stack([Wfc_p[:HID], Wfc_a[:HID]])
  wfb_stack = jnp.stack([Wfc_p[HID:], Wfc_a[HID:]])
  b_stack = jnp.stack([bfc_p, bfc_a]).reshape(NC, 1, OUT)
  return _final_layer(mn2, features, w1_stack, wfa_stack, wfb_stack, b_stack)


# speedup vs baseline: 2.9405x; 1.6547x over previous
import jax
import jax.numpy as jnp
from jax import lax
from jax.experimental import pallas as pl
from jax.experimental.pallas import tpu as pltpu
from jax.experimental.pallas import tpu_sc as plsc

N_P = 5000
NODE_SIZE = 10000
FT = 256
HID = 256
OUT = 256
E = 80000

NC, NS, L = 2, 16, 16
SEG = FT // L
RT = 320
ACC_ROWS = NS * RT
EC = 1024
EPAD = 81920
CAP = 128
FLUSH_AT = CAP - L
KSPM = 3584
IGN = 8191


def _spmm_body(rows_hbm, cols_hbm, w_hbm, table_hbm, out_hbm,
               rbuf, cbuf, wbuf, cstage, wstage, rstage, cstga, cstgb, xbuf,
               acc, spm_table, sem, sem2):
  c = lax.axis_index("c")
  s = lax.axis_index("s")
  lo = s * RT

  off = 0
  for n in (CAP, KSPM // NS - CAP):
    seg = pl.ds(s * (KSPM // NS) + off, n)
    buf = xbuf.at[pl.ds(0, n)]
    pltpu.sync_copy(table_hbm.at[c, seg], buf)
    pltpu.sync_copy(buf, spm_table.at[seg])
    off += n

  zero_i = jnp.zeros((L,), jnp.int32)
  zero_f = jnp.zeros((L,), jnp.float32)

  def zrow(r, carry):
    for si in range(SEG):
      acc[r, pl.ds(si * L, L)] = zero_f
    return carry

  lax.fori_loop(0, RT, zrow, 0)
  for g in range(CAP // L):
    cstage[pl.ds(g * L, L)] = zero_i
    rstage[pl.ds(g * L, L)] = zero_i
    wstage[pl.ds(g * L, L)] = zero_f

  plsc.subcore_barrier()

  def flush():
    def split(g, carry):
      c16 = cstage[pl.ds(g * L, L)]
      in_spm = c16 < KSPM
      cstga[pl.ds(g * L, L)] = jnp.where(in_spm, c16, IGN)
      cstgb[pl.ds(g * L, L)] = jnp.where(in_spm, IGN, c16)
      return carry

    lax.fori_loop(0, CAP // L, split, 0)
    copy_a = pltpu.async_copy(
        spm_table.at[plsc.Indices(cstga, ignored_value=IGN)], xbuf, sem)
    copy_b = pltpu.async_copy(
        table_hbm.at[c].at[plsc.Indices(cstgb, ignored_value=IGN)], xbuf,
        sem2)
    copy_a.wait()
    copy_b.wait()

    def fgroup(g, carry):
      w16 = wstage[pl.ds(g * L, L)]
      r16 = rstage[pl.ds(g * L, L)]
      for l in range(L):
        w = w16[l]
        r = r16[l]
        j = g * L + l

        @plsc.parallel_loop(0, SEG // 2, unroll=4)
        def _(si):
          xw = xbuf[j, pl.ds(si * L, L)]
          xb = plsc.bitcast(xw, jnp.bfloat16)
          va, vb = plsc.unpack(xb, format=plsc.PackFormat.INTERLEAVED,
                               preferred_element_type=jnp.float32)
          plsc.addupdate(acc.at[r, pl.ds(2 * si * L, L)], va * w)
          plsc.addupdate(acc.at[r, pl.ds((2 * si + 1) * L, L)], vb * w)
      wstage[pl.ds(g * L, L)] = zero_f
      return carry

    lax.fori_loop(0, CAP // L, fgroup, 0)

  def chunk(ch, cnt):
    base = ch * EC
    pltpu.sync_copy(rows_hbm.at[c, pl.ds(base, EC)], rbuf)
    pltpu.sync_copy(cols_hbm.at[c, pl.ds(base, EC)], cbuf)
    pltpu.sync_copy(w_hbm.at[c, pl.ds(base, EC)], wbuf)

    def group(g, cnt2):
      row16 = rbuf[pl.ds(g * L, L)]
      m = (row16 >= lo) & (row16 < lo + RT)
      inc = plsc.cumsum(jnp.where(m, 1, 0))
      pos = cnt2 + inc - 1
      plsc.store_scatter(cstage, [pos], cbuf[pl.ds(g * L, L)], mask=m)
      plsc.store_scatter(wstage, [pos], wbuf[pl.ds(g * L, L)], mask=m)
      plsc.store_scatter(rstage, [pos], row16 - lo, mask=m)
      cnt2 = cnt2 + inc[L - 1]

      @pl.when(cnt2 >= FLUSH_AT)
      def _():
        flush()

      return jnp.where(cnt2 >= FLUSH_AT, 0, cnt2)

    return lax.fori_loop(0, EC // L, group, cnt)

  cnt = lax.fori_loop(0, EPAD // EC, chunk, jnp.int32(0))

  @pl.when(cnt > 0)
  def _():
    flush()

  off = 0
  for n in (CAP, CAP, RT - 2 * CAP):
    def orow(r, carry, off=off, n=n):
      for si in range(SEG // 2):
        va = acc[off + r, pl.ds(2 * si * L, L)]
        vb = acc[off + r, pl.ds((2 * si + 1) * L, L)]
        packed = plsc.pack(va, vb, format=plsc.PackFormat.INTERLEAVED)
        xbuf[r, pl.ds(si * L, L)] = plsc.bitcast(packed, jnp.int32)
      return carry

    lax.fori_loop(0, n, orow, 0)
    pltpu.sync_copy(xbuf.at[pl.ds(0, n)], out_hbm.at[c, pl.ds(lo + off, n)])
    off += n


@jax.jit
def _spmm2(table, rows2, cols2, w2):
  mesh = plsc.VectorSubcoreMesh(core_axis_name="c", subcore_axis_name="s")
  return pl.kernel(
      _spmm_body,
      out_type=jax.ShapeDtypeStruct((NC, ACC_ROWS, FT // 2), jnp.int32),
      mesh=mesh,
      compiler_params=pltpu.CompilerParams(needs_layout_passes=False),
      scratch_types=[
          pltpu.VMEM((EC,), jnp.int32),
          pltpu.VMEM((EC,), jnp.int32),
          pltpu.VMEM((EC,), jnp.float32),
          pltpu.VMEM((CAP,), jnp.int32),
          pltpu.VMEM((CAP,), jnp.float32),
          pltpu.VMEM((CAP,), jnp.int32),
          pltpu.VMEM((CAP,), jnp.int32),
          pltpu.VMEM((CAP,), jnp.int32),
          pltpu.VMEM((CAP, FT // 2), jnp.int32),
          pltpu.VMEM((RT, FT), jnp.float32),
          pltpu.VMEM_SHARED((KSPM, FT // 2), jnp.int32),
          pltpu.SemaphoreType.DMA,
          pltpu.SemaphoreType.DMA,
      ],
  )(rows2, cols2, w2, table)


def _gcn_matmul_body(mn_ref, w_ref, out_ref):
  x = mn_ref[0].astype(jnp.float32)
  out_ref[0] = jnp.maximum(
      jnp.dot(x, w_ref[0], preferred_element_type=jnp.float32), 0.0)


BR = 1280


@jax.jit
def _gcn_layer(mn, w_stack):
  return pl.pallas_call(
      _gcn_matmul_body,
      grid=(NC, ACC_ROWS // BR),
      in_specs=[
          pl.BlockSpec((1, BR, FT), lambda c, i: (c, i, 0)),
          pl.BlockSpec((1, FT, HID), lambda c, i: (c, 0, 0)),
      ],
      out_specs=pl.BlockSpec((1, BR, HID), lambda c, i: (c, i, 0)),
      out_shape=jax.ShapeDtypeStruct((NC, ACC_ROWS, HID), jnp.float32),
  )(mn, w_stack)


def _final_body(mn2_ref, feat_ref, w1_ref, wfa_ref, wfb_ref, b_ref, out_ref):
  x = mn2_ref[0].astype(jnp.float32)
  v = jnp.maximum(
      jnp.dot(x, w1_ref[0], preferred_element_type=jnp.float32), 0.0)
  out_ref[...] = (
      jnp.dot(v, wfa_ref[0], preferred_element_type=jnp.float32)
      + jnp.dot(feat_ref[...], wfb_ref[0], preferred_element_type=jnp.float32)
      + b_ref[0])


FR = 1000


@jax.jit
def _final_layer(mn2, features, w1_stack, wfa_stack, wfb_stack, b_stack):
  nb = N_P // FR
  return pl.pallas_call(
      _final_body,
      grid=(NC * nb,),
      in_specs=[
          pl.BlockSpec((1, FR, HID), lambda i: (i // nb, i % nb, 0)),
          pl.BlockSpec((FR, FT), lambda i: (i, 0)),
          pl.BlockSpec((1, HID, HID), lambda i: (i // nb, 0, 0)),
          pl.BlockSpec((1, HID, OUT), lambda i: (i // nb, 0, 0)),
          pl.BlockSpec((1, FT, OUT), lambda i: (i // nb, 0, 0)),
          pl.BlockSpec((1, 1, OUT), lambda i: (i // nb, 0, 0)),
      ],
      out_specs=pl.BlockSpec((FR, OUT), lambda i: (i, 0)),
      out_shape=jax.ShapeDtypeStruct((NODE_SIZE, OUT), jnp.float32),
  )(mn2, features, w1_stack, wfa_stack, wfb_stack, b_stack)


def kernel(features, edge_index_p, edge_weight_p, edge_index_a, edge_weight_a,
           idx_p, idx_a, W0_pa, W0_ap, W1_pa, W1_ap, Wfc_p, bfc_p, Wfc_a,
           bfc_a):
  pad = EPAD - E
  rows2 = jnp.stack([
      jnp.pad(edge_index_p[0], (0, pad)),
      jnp.pad(edge_index_a[0], (0, pad)),
  ])
  cols2 = jnp.stack([
      jnp.pad(edge_index_p[1], (0, pad)),
      jnp.pad(edge_index_a[1], (0, pad)),
  ])
  w2 = jnp.stack([
      jnp.pad(edge_weight_p, (0, pad)),
      jnp.pad(edge_weight_a, (0, pad)),
  ])

  def to_bf16_interleaved(t):
    n = t.shape[:-1]
    b = (t.reshape(*n, FT // 32, 2, L)
         .swapaxes(-2, -1)
         .reshape(*n, FT // 2, 2).astype(jnp.bfloat16))
    return jax.lax.bitcast_convert_type(b, jnp.int32)

  def permute_rows(w):
    return (w.reshape(FT // 32, 2, L, w.shape[-1]).transpose(0, 2, 1, 3)
            .reshape(FT, w.shape[-1]))

  rowpad = ACC_ROWS - N_P
  table_l0 = to_bf16_interleaved(jnp.stack([
      jnp.pad(features[N_P:], ((0, rowpad), (0, 0))),
      jnp.pad(features[:N_P], ((0, rowpad), (0, 0))),
  ]))
  def as_bf16(mn_i32):
    return jax.lax.bitcast_convert_type(mn_i32, jnp.bfloat16).reshape(
        NC, ACC_ROWS, FT)

  mn = as_bf16(_spmm2(table_l0, rows2, cols2, w2))
  w0_stack = jnp.stack([permute_rows(W0_pa), permute_rows(W0_ap)])
  embs1 = _gcn_layer(mn, w0_stack)
  table_l1 = to_bf16_interleaved(embs1[::-1])
  mn2 = as_bf16(_spmm2(table_l1, rows2, cols2, w2))
  w1_stack = jnp.stack([permute_rows(W1_pa), permute_rows(W1_ap)])
  wfa_stack = jnp.stack([Wfc_p[:HID], Wfc_a[:HID]])
  wfb_stack = jnp.stack([Wfc_p[HID:], Wfc_a[HID:]])
  b_stack = jnp.stack([bfc_p, bfc_a]).reshape(NC, 1, OUT)
  return _final_layer(mn2, features, w1_stack, wfa_stack, wfb_stack, b_stack)


# speedup vs baseline: 3.3395x; 1.1357x over previous
import jax
import jax.numpy as jnp
from jax import lax
from jax.experimental import pallas as pl
from jax.experimental.pallas import tpu as pltpu
from jax.experimental.pallas import tpu_sc as plsc

N_P = 5000
NODE_SIZE = 10000
FT = 256
HID = 256
OUT = 256
E = 80000

NC, NS, L = 2, 16, 16
SEG = FT // L
RT = 320
ACC_ROWS = NS * RT
EC = 1024
EPAD = 81920
CAP = 128
FLUSH_AT = CAP - L
KSPM = 3584
IGN = 8191


def _spmm_body(edges_hbm, table_hbm, out_hbm,
               ebuf, cstage, wstage, rstage, cstga, cstgb, xbuf,
               acc, spm_table, sem, sem2):
  c = lax.axis_index("c")
  s = lax.axis_index("s")
  lo = s * RT

  off = 0
  for n in (CAP, KSPM // NS - CAP):
    seg = pl.ds(s * (KSPM // NS) + off, n)
    buf = xbuf.at[pl.ds(0, n)]
    pltpu.sync_copy(table_hbm.at[c, seg], buf)
    pltpu.sync_copy(buf, spm_table.at[seg])
    off += n

  zero_i = jnp.zeros((L,), jnp.int32)
  zero_f = jnp.zeros((L,), jnp.float32)

  def zrow(r, carry):
    for si in range(SEG):
      acc[r, pl.ds(si * L, L)] = zero_f
    return carry

  lax.fori_loop(0, RT, zrow, 0)
  for g in range(CAP // L):
    cstage[pl.ds(g * L, L)] = zero_i
    rstage[pl.ds(g * L, L)] = zero_i
    wstage[pl.ds(g * L, L)] = zero_f

  plsc.subcore_barrier()

  def flush():
    def split(g, carry):
      c16 = cstage[pl.ds(g * L, L)]
      in_spm = c16 < KSPM
      cstga[pl.ds(g * L, L)] = jnp.where(in_spm, c16, IGN)
      cstgb[pl.ds(g * L, L)] = jnp.where(in_spm, IGN, c16)
      return carry

    lax.fori_loop(0, CAP // L, split, 0)
    copy_a = pltpu.async_copy(
        spm_table.at[plsc.Indices(cstga, ignored_value=IGN)], xbuf, sem)
    copy_b = pltpu.async_copy(
        table_hbm.at[c].at[plsc.Indices(cstgb, ignored_value=IGN)], xbuf,
        sem2)
    copy_a.wait()
    copy_b.wait()

    def fgroup(g, carry):
      w16 = wstage[pl.ds(g * L, L)]
      r16 = rstage[pl.ds(g * L, L)]
      for l in range(L):
        w = w16[l]
        r = r16[l]
        j = g * L + l

        @plsc.parallel_loop(0, SEG // 2, unroll=4)
        def _(si):
          xw = xbuf[j, pl.ds(si * L, L)]
          xb = plsc.bitcast(xw, jnp.bfloat16)
          va, vb = plsc.unpack(xb, format=plsc.PackFormat.INTERLEAVED,
                               preferred_element_type=jnp.float32)
          plsc.addupdate(acc.at[r, pl.ds(2 * si * L, L)], va * w)
          plsc.addupdate(acc.at[r, pl.ds((2 * si + 1) * L, L)], vb * w)
      wstage[pl.ds(g * L, L)] = zero_f
      return carry

    lax.fori_loop(0, CAP // L, fgroup, 0)

  def chunk(ch, cnt):
    pltpu.sync_copy(edges_hbm.at[c, pl.ds(ch * 3 * EC, 3 * EC)], ebuf)

    def group(g, cnt2):
      row16 = ebuf[pl.ds(g * L, L)]
      m = (row16 >= lo) & (row16 < lo + RT)
      inc = plsc.cumsum(jnp.where(m, 1, 0))
      pos = cnt2 + inc - 1
      plsc.store_scatter(cstage, [pos], ebuf[pl.ds(EC + g * L, L)], mask=m)
      plsc.store_scatter(wstage, [pos],
                         plsc.bitcast(ebuf[pl.ds(2 * EC + g * L, L)],
                                      jnp.float32), mask=m)
      plsc.store_scatter(rstage, [pos], row16 - lo, mask=m)
      cnt2 = cnt2 + inc[L - 1]

      @pl.when(cnt2 >= FLUSH_AT)
      def _():
        flush()

      return jnp.where(cnt2 >= FLUSH_AT, 0, cnt2)

    return lax.fori_loop(0, EC // L, group, cnt)

  cnt = lax.fori_loop(0, EPAD // EC, chunk, jnp.int32(0))

  @pl.when(cnt > 0)
  def _():
    flush()

  off = 0
  for n in (CAP, CAP, RT - 2 * CAP):
    def orow(r, carry, off=off, n=n):
      for si in range(SEG // 2):
        va = acc[off + r, pl.ds(2 * si * L, L)]
        vb = acc[off + r, pl.ds((2 * si + 1) * L, L)]
        packed = plsc.pack(va, vb, format=plsc.PackFormat.INTERLEAVED)
        xbuf[r, pl.ds(si * L, L)] = plsc.bitcast(packed, jnp.int32)
      return carry

    lax.fori_loop(0, n, orow, 0)
    pltpu.sync_copy(xbuf.at[pl.ds(0, n)], out_hbm.at[c, pl.ds(lo + off, n)])
    off += n


@jax.jit
def _spmm2(table, edges2):
  mesh = plsc.VectorSubcoreMesh(core_axis_name="c", subcore_axis_name="s")
  return pl.kernel(
      _spmm_body,
      out_type=jax.ShapeDtypeStruct((NC, ACC_ROWS, FT // 2), jnp.int32),
      mesh=mesh,
      compiler_params=pltpu.CompilerParams(needs_layout_passes=False),
      scratch_types=[
          pltpu.VMEM((3 * EC,), jnp.int32),
          pltpu.VMEM((CAP,), jnp.int32),
          pltpu.VMEM((CAP,), jnp.float32),
          pltpu.VMEM((CAP,), jnp.int32),
          pltpu.VMEM((CAP,), jnp.int32),
          pltpu.VMEM((CAP,), jnp.int32),
          pltpu.VMEM((CAP, FT // 2), jnp.int32),
          pltpu.VMEM((RT, FT), jnp.float32),
          pltpu.VMEM_SHARED((KSPM, FT // 2), jnp.int32),
          pltpu.SemaphoreType.DMA,
          pltpu.SemaphoreType.DMA,
      ],
  )(edges2, table)


def _gcn_matmul_body(mn_ref, w_ref, out_ref):
  x = mn_ref[0].astype(jnp.float32)
  out_ref[0] = jnp.maximum(
      jnp.dot(x, w_ref[0], preferred_element_type=jnp.float32), 0.0)


BR = 1280


@jax.jit
def _gcn_layer(mn, w_stack):
  return pl.pallas_call(
      _gcn_matmul_body,
      grid=(NC, ACC_ROWS // BR),
      in_specs=[
          pl.BlockSpec((1, BR, FT), lambda c, i: (c, i, 0)),
          pl.BlockSpec((1, FT, HID), lambda c, i: (c, 0, 0)),
      ],
      out_specs=pl.BlockSpec((1, BR, HID), lambda c, i: (c, i, 0)),
      out_shape=jax.ShapeDtypeStruct((NC, ACC_ROWS, HID), jnp.float32),
  )(mn, w_stack)


def _final_body(mn2_ref, feat_ref, w1_ref, wfa_ref, wfb_ref, b_ref, out_ref):
  x = mn2_ref[0].astype(jnp.float32)
  v = jnp.maximum(
      jnp.dot(x, w1_ref[0], preferred_element_type=jnp.float32), 0.0)
  out_ref[...] = (
      jnp.dot(v, wfa_ref[0], preferred_element_type=jnp.float32)
      + jnp.dot(feat_ref[...], wfb_ref[0], preferred_element_type=jnp.float32)
      + b_ref[0])


FR = 1000


@jax.jit
def _final_layer(mn2, features, w1_stack, wfa_stack, wfb_stack, b_stack):
  nb = N_P // FR
  return pl.pallas_call(
      _final_body,
      grid=(NC * nb,),
      in_specs=[
          pl.BlockSpec((1, FR, HID), lambda i: (i // nb, i % nb, 0)),
          pl.BlockSpec((FR, FT), lambda i: (i, 0)),
          pl.BlockSpec((1, HID, HID), lambda i: (i // nb, 0, 0)),
          pl.BlockSpec((1, HID, OUT), lambda i: (i // nb, 0, 0)),
          pl.BlockSpec((1, FT, OUT), lambda i: (i // nb, 0, 0)),
          pl.BlockSpec((1, 1, OUT), lambda i: (i // nb, 0, 0)),
      ],
      out_specs=pl.BlockSpec((FR, OUT), lambda i: (i, 0)),
      out_shape=jax.ShapeDtypeStruct((NODE_SIZE, OUT), jnp.float32),
  )(mn2, features, w1_stack, wfa_stack, wfb_stack, b_stack)


def kernel(features, edge_index_p, edge_weight_p, edge_index_a, edge_weight_a,
           idx_p, idx_a, W0_pa, W0_ap, W1_pa, W1_ap, Wfc_p, bfc_p, Wfc_a,
           bfc_a):
  pad = EPAD - E
  rows2 = jnp.stack([
      jnp.pad(edge_index_p[0], (0, pad)),
      jnp.pad(edge_index_a[0], (0, pad)),
  ])
  cols2 = jnp.stack([
      jnp.pad(edge_index_p[1], (0, pad)),
      jnp.pad(edge_index_a[1], (0, pad)),
  ])
  wbits = jax.lax.bitcast_convert_type(jnp.stack([
      jnp.pad(edge_weight_p, (0, pad)),
      jnp.pad(edge_weight_a, (0, pad)),
  ]), jnp.int32)
  nch = EPAD // EC
  edges2 = jnp.stack([
      rows2.reshape(NC, nch, EC),
      cols2.reshape(NC, nch, EC),
      wbits.reshape(NC, nch, EC),
  ], axis=2).reshape(NC, nch * 3 * EC)

  def to_bf16_interleaved(t):
    n = t.shape[:-1]
    b = (t.reshape(*n, FT // 32, 2, L)
         .swapaxes(-2, -1)
         .reshape(*n, FT // 2, 2).astype(jnp.bfloat16))
    return jax.lax.bitcast_convert_type(b, jnp.int32)

  def permute_rows(w):
    return (w.reshape(FT // 32, 2, L, w.shape[-1]).transpose(0, 2, 1, 3)
            .reshape(FT, w.shape[-1]))

  rowpad = ACC_ROWS - N_P
  table_l0 = to_bf16_interleaved(jnp.stack([
      jnp.pad(features[N_P:], ((0, rowpad), (0, 0))),
      jnp.pad(features[:N_P], ((0, rowpad), (0, 0))),
  ]))
  def as_bf16(mn_i32):
    return jax.lax.bitcast_convert_type(mn_i32, jnp.bfloat16).reshape(
        NC, ACC_ROWS, FT)

  mn = as_bf16(_spmm2(table_l0, edges2))
  w0_stack = jnp.stack([permute_rows(W0_pa), permute_rows(W0_ap)])
  embs1 = _gcn_layer(mn, w0_stack)
  table_l1 = to_bf16_interleaved(embs1[::-1])
  mn2 = as_bf16(_spmm2(table_l1, edges2))
  w1_stack = jnp.stack([permute_rows(W1_pa), permute_rows(W1_ap)])
  wfa_stack = jnp.stack([Wfc_p[:HID], Wfc_a[:HID]])
  wfb_stack = jnp.stack([Wfc_p[HID:], Wfc_a[HID:]])
  b_stack = jnp.stack([bfc_p, bfc_a]).reshape(NC, 1, OUT)
  return _final_layer(mn2, features, w1_stack, wfa_stack, wfb_stack, b_stack)


# speedup vs baseline: 4.3213x; 1.2940x over previous
import jax
import jax.numpy as jnp
from jax import lax
from jax.experimental import pallas as pl
from jax.experimental.pallas import tpu as pltpu
from jax.experimental.pallas import tpu_sc as plsc

N_P = 5000
NODE_SIZE = 10000
FT = 256
HID = 256
OUT = 256
E = 80000

NC, NS, L = 2, 16, 16
SEG = FT // L
RT = 320
ACC_ROWS = NS * RT
EC = 1024
EPAD = 81920
CAP = 128
FLUSH_AT = CAP - L
KSPM = 3584
IGN = 8191


def _spmm_body(edges_hbm, table_hbm, out_hbm,
               ebuf, cstage, wstage, rstage, cstga, cstgb, xbuf,
               acc, spm_table, sem, sem2):
  c = lax.axis_index("c")
  s = lax.axis_index("s")
  lo = s * RT

  off = 0
  for n in (CAP, KSPM // NS - CAP):
    seg = pl.ds(s * (KSPM // NS) + off, n)
    buf = xbuf.at[pl.ds(0, n)]
    pltpu.sync_copy(table_hbm.at[c, seg], buf)
    pltpu.sync_copy(buf, spm_table.at[seg])
    off += n

  zero_i = jnp.zeros((L,), jnp.int32)
  zero_f = jnp.zeros((L,), jnp.float32)

  def zrow(r, carry):
    for si in range(SEG):
      acc[r, pl.ds(si * L, L)] = zero_f
    return carry

  lax.fori_loop(0, RT, zrow, 0)
  for g in range(CAP // L):
    cstage[pl.ds(g * L, L)] = zero_i
    rstage[pl.ds(g * L, L)] = zero_i
    wstage[pl.ds(g * L, L)] = zero_f

  plsc.subcore_barrier()

  def flush():
    def split(g, carry):
      c16 = cstage[pl.ds(g * L, L)]
      in_spm = c16 < KSPM
      cstga[pl.ds(g * L, L)] = jnp.where(in_spm, c16, IGN)
      cstgb[pl.ds(g * L, L)] = jnp.where(in_spm, IGN, c16)
      return carry

    lax.fori_loop(0, CAP // L, split, 0)
    copy_a = pltpu.async_copy(
        spm_table.at[plsc.Indices(cstga, ignored_value=IGN)], xbuf, sem)
    copy_b = pltpu.async_copy(
        table_hbm.at[c].at[plsc.Indices(cstgb, ignored_value=IGN)], xbuf,
        sem2)
    copy_a.wait()
    copy_b.wait()

    def fgroup(g, carry):
      w16 = wstage[pl.ds(g * L, L)]
      r16 = rstage[pl.ds(g * L, L)]
      for l in range(L):
        w = w16[l]
        r = r16[l]
        j = g * L + l

        @plsc.parallel_loop(0, SEG // 2, unroll=8)
        def _(si):
          xw = xbuf[j, pl.ds(si * L, L)]
          xb = plsc.bitcast(xw, jnp.bfloat16)
          va, vb = plsc.unpack(xb, format=plsc.PackFormat.INTERLEAVED,
                               preferred_element_type=jnp.float32)
          plsc.addupdate(acc.at[r, pl.ds(2 * si * L, L)], va * w)
          plsc.addupdate(acc.at[r, pl.ds((2 * si + 1) * L, L)], vb * w)
      wstage[pl.ds(g * L, L)] = zero_f
      return carry

    lax.fori_loop(0, CAP // L, fgroup, 0)

  def chunk(ch, cnt):
    pltpu.sync_copy(edges_hbm.at[c, pl.ds(ch * 3 * EC, 3 * EC)], ebuf)

    def group(g, cnt2):
      row16 = ebuf[pl.ds(g * L, L)]
      m = (row16 >= lo) & (row16 < lo + RT)
      inc = plsc.cumsum(jnp.where(m, 1, 0))
      pos = cnt2 + inc - 1
      plsc.store_scatter(cstage, [pos], ebuf[pl.ds(EC + g * L, L)], mask=m)
      plsc.store_scatter(wstage, [pos],
                         plsc.bitcast(ebuf[pl.ds(2 * EC + g * L, L)],
                                      jnp.float32), mask=m)
      plsc.store_scatter(rstage, [pos], row16 - lo, mask=m)
      cnt2 = cnt2 + inc[L - 1]

      @pl.when(cnt2 >= FLUSH_AT)
      def _():
        flush()

      return jnp.where(cnt2 >= FLUSH_AT, 0, cnt2)

    return lax.fori_loop(0, EC // L, group, cnt)

  cnt = lax.fori_loop(0, EPAD // EC, chunk, jnp.int32(0))

  @pl.when(cnt > 0)
  def _():
    flush()

  off = 0
  for n in (CAP, CAP, RT - 2 * CAP):
    def orow(r, carry, off=off, n=n):
      for si in range(SEG // 2):
        va = acc[off + r, pl.ds(2 * si * L, L)]
        vb = acc[off + r, pl.ds((2 * si + 1) * L, L)]
        packed = plsc.pack(va, vb, format=plsc.PackFormat.INTERLEAVED)
        xbuf[r, pl.ds(si * L, L)] = plsc.bitcast(packed, jnp.int32)
      return carry

    lax.fori_loop(0, n, orow, 0)
    pltpu.sync_copy(xbuf.at[pl.ds(0, n)], out_hbm.at[c, pl.ds(lo + off, n)])
    off += n


@jax.jit
def _spmm2(table, edges2):
  mesh = plsc.VectorSubcoreMesh(core_axis_name="c", subcore_axis_name="s")
  return pl.kernel(
      _spmm_body,
      out_type=jax.ShapeDtypeStruct((NC, ACC_ROWS, FT // 2), jnp.int32),
      mesh=mesh,
      compiler_params=pltpu.CompilerParams(needs_layout_passes=False),
      scratch_types=[
          pltpu.VMEM((3 * EC,), jnp.int32),
          pltpu.VMEM((CAP,), jnp.int32),
          pltpu.VMEM((CAP,), jnp.float32),
          pltpu.VMEM((CAP,), jnp.int32),
          pltpu.VMEM((CAP,), jnp.int32),
          pltpu.VMEM((CAP,), jnp.int32),
          pltpu.VMEM((CAP, FT // 2), jnp.int32),
          pltpu.VMEM((RT, FT), jnp.float32),
          pltpu.VMEM_SHARED((KSPM, FT // 2), jnp.int32),
          pltpu.SemaphoreType.DMA,
          pltpu.SemaphoreType.DMA,
      ],
  )(edges2, table)


def _gcn_matmul_body(mn_ref, w_ref, out_ref):
  x = mn_ref[0].astype(jnp.float32)
  out_ref[0] = jnp.maximum(
      jnp.dot(x, w_ref[0], preferred_element_type=jnp.float32), 0.0)


BR = 1280


@jax.jit
def _gcn_layer(mn, w_stack):
  return pl.pallas_call(
      _gcn_matmul_body,
      grid=(NC, ACC_ROWS // BR),
      in_specs=[
          pl.BlockSpec((1, BR, FT), lambda c, i: (c, i, 0)),
          pl.BlockSpec((1, FT, HID), lambda c, i: (c, 0, 0)),
      ],
      out_specs=pl.BlockSpec((1, BR, HID), lambda c, i: (c, i, 0)),
      out_shape=jax.ShapeDtypeStruct((NC, ACC_ROWS, HID), jnp.float32),
  )(mn, w_stack)


def _final_body(mn2_ref, feat_ref, w1_ref, wfa_ref, wfb_ref, b_ref, out_ref):
  x = mn2_ref[0].astype(jnp.float32)
  v = jnp.maximum(
      jnp.dot(x, w1_ref[0], preferred_element_type=jnp.float32), 0.0)
  out_ref[...] = (
      jnp.dot(v, wfa_ref[0], preferred_element_type=jnp.float32)
      + jnp.dot(feat_ref[...], wfb_ref[0], preferred_element_type=jnp.float32)
      + b_ref[0])


FR = 1000


@jax.jit
def _final_layer(mn2, features, w1_stack, wfa_stack, wfb_stack, b_stack):
  nb = N_P // FR
  return pl.pallas_call(
      _final_body,
      grid=(NC * nb,),
      in_specs=[
          pl.BlockSpec((1, FR, HID), lambda i: (i // nb, i % nb, 0)),
          pl.BlockSpec((FR, FT), lambda i: (i, 0)),
          pl.BlockSpec((1, HID, HID), lambda i: (i // nb, 0, 0)),
          pl.BlockSpec((1, HID, OUT), lambda i: (i // nb, 0, 0)),
          pl.BlockSpec((1, FT, OUT), lambda i: (i // nb, 0, 0)),
          pl.BlockSpec((1, 1, OUT), lambda i: (i // nb, 0, 0)),
      ],
      out_specs=pl.BlockSpec((FR, OUT), lambda i: (i, 0)),
      out_shape=jax.ShapeDtypeStruct((NODE_SIZE, OUT), jnp.float32),
  )(mn2, features, w1_stack, wfa_stack, wfb_stack, b_stack)


def kernel(features, edge_index_p, edge_weight_p, edge_index_a, edge_weight_a,
           idx_p, idx_a, W0_pa, W0_ap, W1_pa, W1_ap, Wfc_p, bfc_p, Wfc_a,
           bfc_a):
  pad = EPAD - E
  rows2 = jnp.stack([
      jnp.pad(edge_index_p[0], (0, pad)),
      jnp.pad(edge_index_a[0], (0, pad)),
  ])
  cols2 = jnp.stack([
      jnp.pad(edge_index_p[1], (0, pad)),
      jnp.pad(edge_index_a[1], (0, pad)),
  ])
  wbits = jax.lax.bitcast_convert_type(jnp.stack([
      jnp.pad(edge_weight_p, (0, pad)),
      jnp.pad(edge_weight_a, (0, pad)),
  ]), jnp.int32)
  nch = EPAD // EC
  edges2 = jnp.stack([
      rows2.reshape(NC, nch, EC),
      cols2.reshape(NC, nch, EC),
      wbits.reshape(NC, nch, EC),
  ], axis=2).reshape(NC, nch * 3 * EC)

  def to_bf16_interleaved(t):
    n = t.shape[:-1]
    b = (t.reshape(*n, FT // 32, 2, L)
         .swapaxes(-2, -1)
         .reshape(*n, FT // 2, 2).astype(jnp.bfloat16))
    return jax.lax.bitcast_convert_type(b, jnp.int32)

  def permute_rows(w):
    return (w.reshape(FT // 32, 2, L, w.shape[-1]).transpose(0, 2, 1, 3)
            .reshape(FT, w.shape[-1]))

  rowpad = ACC_ROWS - N_P
  table_l0 = to_bf16_interleaved(jnp.stack([
      jnp.pad(features[N_P:], ((0, rowpad), (0, 0))),
      jnp.pad(features[:N_P], ((0, rowpad), (0, 0))),
  ]))
  def as_bf16(mn_i32):
    return jax.lax.bitcast_convert_type(mn_i32, jnp.bfloat16).reshape(
        NC, ACC_ROWS, FT)

  mn = as_bf16(_spmm2(table_l0, edges2))
  w0_stack = jnp.stack([permute_rows(W0_pa), permute_rows(W0_ap)])
  embs1 = _gcn_layer(mn, w0_stack)
  table_l1 = to_bf16_interleaved(embs1[::-1])
  mn2 = as_bf16(_spmm2(table_l1, edges2))
  w1_stack = jnp.stack([permute_rows(W1_pa), permute_rows(W1_ap)])
  wfa_stack = jnp.stack([Wfc_p[:HID], Wfc_a[:HID]])
  wfb_stack = jnp.stack([Wfc_p[HID:], Wfc_a[HID:]])
  b_stack = jnp.stack([bfc_p, bfc_a]).reshape(NC, 1, OUT)
  return _final_layer(mn2, features, w1_stack, wfa_stack, wfb_stack, b_stack)
